# Initial kernel scaffold; baseline (speedup 1.0000x reference)
import jax
import jax.numpy as jnp
from jax.experimental import pallas as pl


def kernel(h, rel_x, r_feat, edge_feat, edge_index, e_w, id3_i, id3_j, id3_k, edgeid_ki, edgeid_ji, edgeid_kj, xk_W1, xk_b1, xk_g, xk_be, xk_W2, xk_b2, xv_W1, xv_b1, xv_g, xv_be, xv_W2, xv_b2, xq_W1, xq_b1, xq_g, xq_be, xq_W2, xq_b2, ew_W, ew_b):
    raise NotImplementedError("write your pallas kernel here")


# speedup vs baseline: 12.6275x; 12.6275x over previous
import functools

import jax
import jax.numpy as jnp
import numpy as np
from jax.experimental import pallas as pl
from jax.experimental.pallas import tpu as pltpu
from jax.experimental.pallas import tpu_sc as plsc

N = 10000
E = 160000
T = 320000
NH = 16
HID = 128
B = 256
NB = T // B
GW = 128
EW = 128


def _q_body(h_ref, W1_ref, b1_ref, g_ref, be_ref, W2_ref, b2_ref, o_ref):
    y = jnp.dot(h_ref[...], W1_ref[...], preferred_element_type=jnp.float32)
    y = y + b1_ref[...]
    mu = jnp.mean(y, axis=1, keepdims=True)
    var = jnp.mean((y - mu) * (y - mu), axis=1, keepdims=True)
    y = (y - mu) * jax.lax.rsqrt(var + 1e-5) * g_ref[...] + be_ref[...]
    y = jnp.maximum(y, 0.0)
    o_ref[...] = jnp.dot(y, W2_ref[...], preferred_element_type=jnp.float32) + b2_ref[...]


def _main_body(idc_ref, idr_ref, eji_ref, eki_ref, ekj_ref, hi_ref, hj_ref,
               hk_ref, qi_ref, Wbig_ref, b1_ref, g_ref, be_ref, Wk2_ref,
               bk2_ref, Wv2_ref, bv2_ref, ewW_ref, ewb_ref, G_ref, out_ref,
               accM, accS, accU, Sseg, Cnt):
    pid = pl.program_id(0)

    @pl.when(pid == 0)
    def _init():
        accM[...] = jnp.full(accM.shape, -1e30, jnp.float32)
        accS[...] = jnp.zeros(accS.shape, jnp.float32)
        accU[...] = jnp.zeros(accU.shape, jnp.float32)

    erw_ji = eji_ref[...]
    erw_ki = eki_ref[...]
    erw_kj = ekj_ref[...]
    hi = hi_ref[...]
    hj = hj_ref[...]
    hk = hk_ref[...]

    X = jnp.concatenate([erw_ji[:, :32], erw_ki[:, :32], erw_kj[:, :32],
                         hi, hj, hk], axis=1)
    Y = jnp.dot(X, Wbig_ref[...], preferred_element_type=jnp.float32)
    Y = Y + b1_ref[...]
    g = g_ref[...]
    be = be_ref[...]
    subs = []
    for t in range(4):
        y = Y[:, 128 * t:128 * (t + 1)]
        mu = jnp.mean(y, axis=1, keepdims=True)
        var = jnp.mean((y - mu) * (y - mu), axis=1, keepdims=True)
        y = (y - mu) * jax.lax.rsqrt(var + 1e-5) * g[:, 128 * t:128 * (t + 1)]
        y = y + be[:, 128 * t:128 * (t + 1)]
        subs.append(jnp.maximum(y, 0.0))

    Wk2 = Wk2_ref[...]
    Wv2 = Wv2_ref[...]
    k1 = jnp.dot(subs[0], Wk2, preferred_element_type=jnp.float32) + bk2_ref[...]
    k2 = jnp.dot(subs[1], Wk2, preferred_element_type=jnp.float32) + bk2_ref[...]
    v1 = jnp.dot(subs[2], Wv2, preferred_element_type=jnp.float32) + bv2_ref[...]
    v2 = jnp.dot(subs[3], Wv2, preferred_element_type=jnp.float32) + bv2_ref[...]

    q = qi_ref[...]
    G = G_ref[...]
    inv = np.float32(1.0 / np.sqrt(8.0))
    e1 = jnp.dot(k1 * q, G, preferred_element_type=jnp.float32) * inv
    e2 = jnp.dot(k2 * q, G, preferred_element_type=jnp.float32) * inv

    ewW = ewW_ref[...]
    w_ji = jax.nn.sigmoid(
        jnp.sum(erw_ji[:, 16:32] * ewW, axis=1, keepdims=True) + ewb_ref[...])
    w_ki = jax.nn.sigmoid(
        jnp.sum(erw_ki[:, 16:32] * ewW, axis=1, keepdims=True) + ewb_ref[...])

    P = (v1[:, :16] + v2[:, :16]) * w_ji * 0.5
    Qv = (v1[:, 16:] + v2[:, 16:]) * w_ki * 0.5
    vcs = []
    for c in range(3):
        rj = erw_ji[:, 32 + c:33 + c]
        rk = erw_ki[:, 32 + c:33 + c]
        vcs.append(P * rj + Qv * rk)

    M1 = jnp.max(e1, axis=0, keepdims=True)
    M2 = jnp.max(e2, axis=0, keepdims=True)
    ex1 = jnp.exp(e1 - M1)
    ex2 = jnp.exp(e2 - M2)

    cols = jnp.concatenate(
        [ex1, ex2,
         ex1 * vcs[0], ex1 * vcs[1], ex1 * vcs[2],
         ex2 * vcs[0], ex2 * vcs[1], ex2 * vcs[2]], axis=1)

    ids_col = idc_ref[...]
    ids_row = idr_ref[...][0]
    Eq = (ids_col == ids_row).astype(jnp.float32)
    Sseg[...] = jnp.dot(Eq, cols, preferred_element_type=jnp.float32)
    Cnt[...] = jnp.sum(Eq, axis=1, keepdims=True)

    Mblk = jnp.concatenate([M1, M2], axis=1)

    def _cond(p):
        return p < B

    def _merge(p):
        n = idc_ref[p, 0]
        srow = Sseg[pl.ds(p, 1), :]
        cnt = Cnt[p, 0].astype(jnp.int32)
        mold = accM[pl.ds(n, 1), :]
        mnew = jnp.maximum(mold, Mblk)
        fa = jnp.exp(mold - mnew)
        fb = jnp.exp(Mblk - mnew)
        sold = accS[pl.ds(n, 1), :]
        accS[pl.ds(n, 1), :] = sold * fa + srow[:, 0:32] * fb
        fa1 = fa[:, 0:16]
        fa2 = fa[:, 16:32]
        fb1 = fb[:, 0:16]
        fb2 = fb[:, 16:32]
        fa96 = jnp.concatenate([fa1, fa1, fa1, fa2, fa2, fa2], axis=1)
        fb96 = jnp.concatenate([fb1, fb1, fb1, fb2, fb2, fb2], axis=1)
        uold = accU[pl.ds(n, 1), :]
        accU[pl.ds(n, 1), :] = uold * fa96 + srow[:, 32:128] * fb96
        accM[pl.ds(n, 1), :] = mnew
        return p + cnt

    jax.lax.while_loop(_cond, _merge, jnp.int32(0))

    @pl.when(pid == NB - 1)
    def _fin():
        d1 = accS[:, 0:16] + 1e-16
        d2 = accS[:, 16:32] + 1e-16
        outs = []
        for c in range(3):
            t = accU[:, 16 * c:16 * c + 16] / d1 \
                + accU[:, 48 + 16 * c:48 + 16 * c + 16] / d2
            outs.append(jnp.sum(t, axis=1, keepdims=True) * np.float32(0.5 / 16.0))
        out_ref[...] = jnp.concatenate(outs, axis=1)


def _sc_gather(table, idx_list):
    info = pltpu.get_tpu_info().sparse_core
    mesh = plsc.VectorSubcoreMesh(core_axis_name="core",
                                  subcore_axis_name="subcore",
                                  num_cores=info.num_cores,
                                  num_subcores=info.num_subcores)
    n_idx = len(idx_list)
    width = table.shape[1]
    outs = tuple(jax.ShapeDtypeStruct((T, width), table.dtype)
                 for _ in range(n_idx))

    @functools.partial(pl.kernel, out_type=outs, mesh=mesh)
    def kern(tab_hbm, *refs):
        idx_refs = refs[:n_idx]
        out_refs = refs[n_idx:]

        def body(*bufs):
            ibufs = bufs[:n_idx]
            obufs = bufs[n_idx:]
            for ib, ob in zip(ibufs, obufs):
                pltpu.sync_copy(tab_hbm.at[ib.at[0]], ob)

        pltpu.emit_pipeline(
            body,
            grid=(T // GW,),
            in_specs=[pl.BlockSpec((1, GW), lambda i: (0, i))] * n_idx,
            out_specs=[pl.BlockSpec((GW, width), lambda i: (i, 0))] * n_idx,
            core_axis_name="subcore",
            dimension_semantics=(pltpu.PARALLEL,),
        )(*idx_refs, *out_refs)

    res = kern(table, *[ix.reshape(1, T) for ix in idx_list])
    return tuple(res) if isinstance(res, (tuple, list)) else (res,)


def _perm_ikj(W):
    return jnp.concatenate([W[32:64], W[0:32], W[64:96],
                            W[96:224], W[352:480], W[224:352]], axis=0)


def kernel(h, rel_x, r_feat, edge_feat, edge_index, e_w, id3_i, id3_j, id3_k,
           edgeid_ki, edgeid_ji, edgeid_kj,
           xk_W1, xk_b1, xk_g, xk_be, xk_W2, xk_b2,
           xv_W1, xv_b1, xv_g, xv_be, xv_W2, xv_b2,
           xq_W1, xq_b1, xq_g, xq_be, xq_W2, xq_b2,
           ew_W, ew_b):
    f32 = jnp.float32

    erw = jnp.concatenate(
        [edge_feat, r_feat, rel_x, jnp.zeros((E, EW - 35), f32)], axis=1)

    qtab = pl.pallas_call(
        _q_body,
        grid=(5,),
        in_specs=[pl.BlockSpec((2000, 128), lambda b: (b, 0)),
                  pl.BlockSpec((128, 128), lambda b: (0, 0)),
                  pl.BlockSpec((1, 128), lambda b: (0, 0)),
                  pl.BlockSpec((1, 128), lambda b: (0, 0)),
                  pl.BlockSpec((1, 128), lambda b: (0, 0)),
                  pl.BlockSpec((128, 128), lambda b: (0, 0)),
                  pl.BlockSpec((1, 128), lambda b: (0, 0))],
        out_specs=pl.BlockSpec((2000, 128), lambda b: (b, 0)),
        out_shape=jax.ShapeDtypeStruct((N, 128), f32),
    )(h, xq_W1, xq_b1.reshape(1, 128), xq_g.reshape(1, 128),
      xq_be.reshape(1, 128), xq_W2, xq_b2.reshape(1, 128))

    g_eji, g_eki, g_ekj = _sc_gather(erw, [edgeid_ji, edgeid_ki, edgeid_kj])
    g_hi, g_hj, g_hk = _sc_gather(h, [id3_i, id3_j, id3_k])
    (g_q,) = _sc_gather(qtab, [id3_i])

    Wbig = jnp.concatenate(
        [xk_W1, _perm_ikj(xk_W1), xv_W1, _perm_ikj(xv_W1)], axis=1)
    b1 = jnp.concatenate([xk_b1, xk_b1, xv_b1, xv_b1]).reshape(1, 512)
    g512 = jnp.concatenate([xk_g, xk_g, xv_g, xv_g]).reshape(1, 512)
    be512 = jnp.concatenate([xk_be, xk_be, xv_be, xv_be]).reshape(1, 512)
    Ghead = (jnp.arange(128)[:, None] // 8
             == jnp.arange(16)[None, :]).astype(f32)

    idc = id3_i.reshape(T, 1)
    idr = id3_i.reshape(NB, 1, B)

    bspec_t48 = pl.BlockSpec((B, EW), lambda b: (b, 0))
    bspec_t128 = pl.BlockSpec((B, 128), lambda b: (b, 0))
    const = lambda shape: pl.BlockSpec(shape, lambda b: tuple(0 for _ in shape))

    out = pl.pallas_call(
        _main_body,
        grid=(NB,),
        in_specs=[pl.BlockSpec((B, 1), lambda b: (b, 0)),
                  pl.BlockSpec((1, 1, B), lambda b: (b, 0, 0)),
                  bspec_t48, bspec_t48, bspec_t48,
                  bspec_t128, bspec_t128, bspec_t128, bspec_t128,
                  const((480, 512)), const((1, 512)), const((1, 512)),
                  const((1, 512)), const((128, 128)), const((1, 128)),
                  const((128, 32)), const((1, 32)), const((1, 16)),
                  const((1, 1)), const((128, 16))],
        out_specs=pl.BlockSpec((N, 3), lambda b: (0, 0)),
        out_shape=jax.ShapeDtypeStruct((N, 3), f32),
        scratch_shapes=[pltpu.VMEM((N, 32), f32),
                        pltpu.VMEM((N, 32), f32),
                        pltpu.VMEM((N, 96), f32),
                        pltpu.VMEM((B, 128), f32),
                        pltpu.VMEM((B, 1), f32)],
    )(idc, idr, g_eji, g_eki, g_ekj, g_hi, g_hj, g_hk, g_q,
      Wbig, b1, g512, be512, xk_W2, xk_b2.reshape(1, 128),
      xv_W2, xv_b2.reshape(1, 32), ew_W.reshape(1, 16), ew_b.reshape(1, 1),
      Ghead)

    return out


# speedup vs baseline: 13.1388x; 1.0405x over previous
import functools

import jax
import jax.numpy as jnp
import numpy as np
from jax.experimental import pallas as pl
from jax.experimental.pallas import tpu as pltpu
from jax.experimental.pallas import tpu_sc as plsc

N = 10000
E = 160000
T = 320000
NH = 16
HID = 128
B = 512
NB = T // B
GW = 128
EW = 128


def _q_body(h_ref, W1_ref, b1_ref, g_ref, be_ref, W2_ref, b2_ref, o_ref):
    y = jnp.dot(h_ref[...], W1_ref[...], preferred_element_type=jnp.float32)
    y = y + b1_ref[...]
    mu = jnp.mean(y, axis=1, keepdims=True)
    var = jnp.mean((y - mu) * (y - mu), axis=1, keepdims=True)
    y = (y - mu) * jax.lax.rsqrt(var + 1e-5) * g_ref[...] + be_ref[...]
    y = jnp.maximum(y, 0.0)
    o_ref[...] = jnp.dot(y, W2_ref[...], preferred_element_type=jnp.float32) + b2_ref[...]


def _main_body(idc_ref, idr_ref, eji_ref, eki_ref, ekj_ref, hj_ref,
               hk_ref, hqi_ref, Wbig_ref, b1_ref, g_ref, be_ref, Wk2_ref,
               bk2_ref, Wv2_ref, bv2_ref, ewW_ref, ewb_ref, G_ref, out_ref,
               accM, accS, accU, Sseg, Cnt):
    pid = pl.program_id(0)

    @pl.when(pid == 0)
    def _init():
        accM[...] = jnp.full(accM.shape, -1e30, jnp.float32)
        accS[...] = jnp.zeros(accS.shape, jnp.float32)
        accU[...] = jnp.zeros(accU.shape, jnp.float32)

    erw_ji = eji_ref[...]
    erw_ki = eki_ref[...]
    erw_kj = ekj_ref[...]
    hqi = hqi_ref[...]
    hi = hqi[:, :128]
    hj = hj_ref[...]
    hk = hk_ref[...]

    X = jnp.concatenate([erw_ji[:, :32], erw_ki[:, :32], erw_kj[:, :32],
                         hi, hj, hk], axis=1)
    Y = jnp.dot(X, Wbig_ref[...], preferred_element_type=jnp.float32)
    Y = Y + b1_ref[...]
    g = g_ref[...]
    be = be_ref[...]
    subs = []
    for t in range(4):
        y = Y[:, 128 * t:128 * (t + 1)]
        mu = jnp.mean(y, axis=1, keepdims=True)
        var = jnp.mean((y - mu) * (y - mu), axis=1, keepdims=True)
        y = (y - mu) * jax.lax.rsqrt(var + 1e-5) * g[:, 128 * t:128 * (t + 1)]
        y = y + be[:, 128 * t:128 * (t + 1)]
        subs.append(jnp.maximum(y, 0.0))

    Wk2 = Wk2_ref[...]
    Wv2 = Wv2_ref[...]
    k1 = jnp.dot(subs[0], Wk2, preferred_element_type=jnp.float32) + bk2_ref[...]
    k2 = jnp.dot(subs[1], Wk2, preferred_element_type=jnp.float32) + bk2_ref[...]
    v1 = jnp.dot(subs[2], Wv2, preferred_element_type=jnp.float32) + bv2_ref[...]
    v2 = jnp.dot(subs[3], Wv2, preferred_element_type=jnp.float32) + bv2_ref[...]

    q = hqi[:, 128:]
    G = G_ref[...]
    inv = np.float32(1.0 / np.sqrt(8.0))
    e1 = jnp.dot(k1 * q, G, preferred_element_type=jnp.float32) * inv
    e2 = jnp.dot(k2 * q, G, preferred_element_type=jnp.float32) * inv

    ewW = ewW_ref[...]
    w_ji = jax.nn.sigmoid(
        jnp.sum(erw_ji[:, 16:32] * ewW, axis=1, keepdims=True) + ewb_ref[...])
    w_ki = jax.nn.sigmoid(
        jnp.sum(erw_ki[:, 16:32] * ewW, axis=1, keepdims=True) + ewb_ref[...])

    P = (v1[:, :16] + v2[:, :16]) * w_ji * 0.5
    Qv = (v1[:, 16:] + v2[:, 16:]) * w_ki * 0.5
    vcs = []
    for c in range(3):
        rj = erw_ji[:, 32 + c:33 + c]
        rk = erw_ki[:, 32 + c:33 + c]
        vcs.append(P * rj + Qv * rk)

    M1 = jnp.max(e1, axis=0, keepdims=True)
    M2 = jnp.max(e2, axis=0, keepdims=True)
    ex1 = jnp.exp(e1 - M1)
    ex2 = jnp.exp(e2 - M2)

    cols = jnp.concatenate(
        [ex1, ex2,
         ex1 * vcs[0], ex1 * vcs[1], ex1 * vcs[2],
         ex2 * vcs[0], ex2 * vcs[1], ex2 * vcs[2]], axis=1)

    ids_col = idc_ref[...]
    ids_row = idr_ref[...][0]
    Eq = (ids_col == ids_row).astype(jnp.float32)
    Sseg[...] = jnp.dot(Eq, cols, preferred_element_type=jnp.float32)
    Cnt[...] = jnp.sum(Eq, axis=1, keepdims=True)

    Mblk = jnp.concatenate([M1, M2], axis=1)

    def _cond(p):
        return p < B

    def _merge(p):
        n = idc_ref[p, 0]
        srow = Sseg[pl.ds(p, 1), :]
        cnt = Cnt[p, 0].astype(jnp.int32)
        mold = accM[pl.ds(n, 1), :]
        mnew = jnp.maximum(mold, Mblk)
        fa = jnp.exp(mold - mnew)
        fb = jnp.exp(Mblk - mnew)
        sold = accS[pl.ds(n, 1), :]
        accS[pl.ds(n, 1), :] = sold * fa + srow[:, 0:32] * fb
        fa1 = fa[:, 0:16]
        fa2 = fa[:, 16:32]
        fb1 = fb[:, 0:16]
        fb2 = fb[:, 16:32]
        fa96 = jnp.concatenate([fa1, fa1, fa1, fa2, fa2, fa2], axis=1)
        fb96 = jnp.concatenate([fb1, fb1, fb1, fb2, fb2, fb2], axis=1)
        uold = accU[pl.ds(n, 1), :]
        accU[pl.ds(n, 1), :] = uold * fa96 + srow[:, 32:128] * fb96
        accM[pl.ds(n, 1), :] = mnew
        return p + cnt

    jax.lax.while_loop(_cond, _merge, jnp.int32(0))

    @pl.when(pid == NB - 1)
    def _fin():
        d1 = accS[:, 0:16] + 1e-16
        d2 = accS[:, 16:32] + 1e-16
        outs = []
        for c in range(3):
            t = accU[:, 16 * c:16 * c + 16] / d1 \
                + accU[:, 48 + 16 * c:48 + 16 * c + 16] / d2
            outs.append(jnp.sum(t, axis=1, keepdims=True) * np.float32(0.5 / 16.0))
        out_ref[...] = jnp.concatenate(outs, axis=1)


def _sc_gather(pairs):
    info = pltpu.get_tpu_info().sparse_core
    mesh = plsc.VectorSubcoreMesh(core_axis_name="core",
                                  subcore_axis_name="subcore",
                                  num_cores=info.num_cores,
                                  num_subcores=info.num_subcores)
    tables = []
    tab_ids = []
    for tab, _ in pairs:
        for ti, t in enumerate(tables):
            if t is tab:
                tab_ids.append(ti)
                break
        else:
            tab_ids.append(len(tables))
            tables.append(tab)
    n = len(pairs)
    n_tab = len(tables)
    outs = tuple(jax.ShapeDtypeStruct((T, tab.shape[1]), tab.dtype)
                 for tab, _ in pairs)

    @functools.partial(pl.kernel, out_type=outs, mesh=mesh)
    def kern(*refs):
        tab_refs = refs[:n_tab]
        idx_refs = refs[n_tab:n_tab + n]
        out_refs = refs[n_tab + n:]

        def body(*bufs):
            ibufs = bufs[:n]
            obufs = bufs[n:]
            for ti, ib, ob in zip(tab_ids, ibufs, obufs):
                pltpu.sync_copy(tab_refs[ti].at[ib.at[0]], ob)

        pltpu.emit_pipeline(
            body,
            grid=(T // GW,),
            in_specs=[pl.BlockSpec((1, GW), lambda i: (0, i))] * n,
            out_specs=[pl.BlockSpec((GW, tab.shape[1]), lambda i: (i, 0))
                       for tab, _ in pairs],
            core_axis_name="subcore",
            dimension_semantics=(pltpu.PARALLEL,),
        )(*idx_refs, *out_refs)

    res = kern(*tables, *[ix.reshape(1, T) for _, ix in pairs])
    return tuple(res) if isinstance(res, (tuple, list)) else (res,)


def _perm_ikj(W):
    return jnp.concatenate([W[32:64], W[0:32], W[64:96],
                            W[96:224], W[352:480], W[224:352]], axis=0)


def kernel(h, rel_x, r_feat, edge_feat, edge_index, e_w, id3_i, id3_j, id3_k,
           edgeid_ki, edgeid_ji, edgeid_kj,
           xk_W1, xk_b1, xk_g, xk_be, xk_W2, xk_b2,
           xv_W1, xv_b1, xv_g, xv_be, xv_W2, xv_b2,
           xq_W1, xq_b1, xq_g, xq_be, xq_W2, xq_b2,
           ew_W, ew_b):
    f32 = jnp.float32

    erw = jnp.concatenate(
        [edge_feat, r_feat, rel_x, jnp.zeros((E, EW - 35), f32)], axis=1)

    qtab = pl.pallas_call(
        _q_body,
        grid=(5,),
        in_specs=[pl.BlockSpec((2000, 128), lambda b: (b, 0)),
                  pl.BlockSpec((128, 128), lambda b: (0, 0)),
                  pl.BlockSpec((1, 128), lambda b: (0, 0)),
                  pl.BlockSpec((1, 128), lambda b: (0, 0)),
                  pl.BlockSpec((1, 128), lambda b: (0, 0)),
                  pl.BlockSpec((128, 128), lambda b: (0, 0)),
                  pl.BlockSpec((1, 128), lambda b: (0, 0))],
        out_specs=pl.BlockSpec((2000, 128), lambda b: (b, 0)),
        out_shape=jax.ShapeDtypeStruct((N, 128), f32),
    )(h, xq_W1, xq_b1.reshape(1, 128), xq_g.reshape(1, 128),
      xq_be.reshape(1, 128), xq_W2, xq_b2.reshape(1, 128))

    hq = jnp.concatenate([h, qtab], axis=1)
    g_eji, g_eki, g_ekj = _sc_gather(
        [(erw, edgeid_ji), (erw, edgeid_ki), (erw, edgeid_kj)])
    g_hj, g_hk = _sc_gather([(h, id3_j), (h, id3_k)])
    (g_hqi,) = _sc_gather([(hq, id3_i)])

    Wbig = jnp.concatenate(
        [xk_W1, _perm_ikj(xk_W1), xv_W1, _perm_ikj(xv_W1)], axis=1)
    b1 = jnp.concatenate([xk_b1, xk_b1, xv_b1, xv_b1]).reshape(1, 512)
    g512 = jnp.concatenate([xk_g, xk_g, xv_g, xv_g]).reshape(1, 512)
    be512 = jnp.concatenate([xk_be, xk_be, xv_be, xv_be]).reshape(1, 512)
    Ghead = (jnp.arange(128)[:, None] // 8
             == jnp.arange(16)[None, :]).astype(f32)

    idc = id3_i.reshape(T, 1)
    idr = id3_i.reshape(NB, 1, B)

    bspec_te = pl.BlockSpec((B, EW), lambda b: (b, 0))
    bspec_t128 = pl.BlockSpec((B, 128), lambda b: (b, 0))
    const = lambda shape: pl.BlockSpec(shape, lambda b: tuple(0 for _ in shape))

    out = pl.pallas_call(
        _main_body,
        grid=(NB,),
        in_specs=[pl.BlockSpec((B, 1), lambda b: (b, 0)),
                  pl.BlockSpec((1, 1, B), lambda b: (b, 0, 0)),
                  bspec_te, bspec_te, bspec_te,
                  bspec_t128, bspec_t128,
                  pl.BlockSpec((B, 256), lambda b: (b, 0)),
                  const((480, 512)), const((1, 512)), const((1, 512)),
                  const((1, 512)), const((128, 128)), const((1, 128)),
                  const((128, 32)), const((1, 32)), const((1, 16)),
                  const((1, 1)), const((128, 16))],
        out_specs=pl.BlockSpec((N, 3), lambda b: (0, 0)),
        out_shape=jax.ShapeDtypeStruct((N, 3), f32),
        scratch_shapes=[pltpu.VMEM((N, 32), f32),
                        pltpu.VMEM((N, 32), f32),
                        pltpu.VMEM((N, 96), f32),
                        pltpu.VMEM((B, 128), f32),
                        pltpu.VMEM((B, 1), f32)],
    )(idc, idr, g_eji, g_eki, g_ekj, g_hj, g_hk, g_hqi,
      Wbig, b1, g512, be512, xk_W2, xk_b2.reshape(1, 128),
      xv_W2, xv_b2.reshape(1, 32), ew_W.reshape(1, 16), ew_b.reshape(1, 1),
      Ghead)

    return out


# speedup vs baseline: 15.9072x; 1.2107x over previous
import functools

import jax
import jax.numpy as jnp
import numpy as np
from jax.experimental import pallas as pl
from jax.experimental.pallas import tpu as pltpu
from jax.experimental.pallas import tpu_sc as plsc

N = 10000
E = 160000
T = 320000
NH = 16
HID = 128
B = 512
NB = T // B
GW = 128
EW = 128


def _q_body(h_ref, W1_ref, b1_ref, g_ref, be_ref, W2_ref, b2_ref, o_ref):
    y = jnp.dot(h_ref[...], W1_ref[...], preferred_element_type=jnp.float32)
    y = y + b1_ref[...]
    mu = jnp.mean(y, axis=1, keepdims=True)
    var = jnp.mean((y - mu) * (y - mu), axis=1, keepdims=True)
    y = (y - mu) * jax.lax.rsqrt(var + 1e-5) * g_ref[...] + be_ref[...]
    y = jnp.maximum(y, 0.0)
    o_ref[...] = jnp.dot(y, W2_ref[...], preferred_element_type=jnp.float32) + b2_ref[...]


def _main_body(first, *refs):
    if first:
        (idc_ref, idr_ref, eji_ref, eki_ref, ekj_ref, hj_ref, hk_ref,
         hqi_ref, Wbig_ref, b1_ref, g_ref, be_ref, Wk2_ref, bk2_ref,
         Wv2_ref, bv2_ref, ewW_ref, ewb_ref, G_ref,
         accM, accS, accU, Sseg, Cnt) = refs
        accMi = accSi = accUi = None
    else:
        (accMi, accSi, accUi, idc_ref, idr_ref, eji_ref, eki_ref, ekj_ref,
         hj_ref, hk_ref, hqi_ref, Wbig_ref, b1_ref, g_ref, be_ref, Wk2_ref,
         bk2_ref, Wv2_ref, bv2_ref, ewW_ref, ewb_ref, G_ref,
         accM, accS, accU, Sseg, Cnt) = refs
    pid = pl.program_id(0)

    @pl.when(pid == 0)
    def _init():
        if first:
            accM[...] = jnp.full((N, 32), -1e30, jnp.float32)
            accS[...] = jnp.zeros((N, 32), jnp.float32)
            accU[...] = jnp.zeros((N, 96), jnp.float32)
        else:
            accM[...] = accMi[...]
            accS[...] = accSi[...]
            accU[...] = accUi[...]

    erw_ji = eji_ref[...]
    erw_ki = eki_ref[...]
    erw_kj = ekj_ref[...]
    hqi = hqi_ref[...]
    hi = hqi[:, :128]
    hj = hj_ref[...]
    hk = hk_ref[...]

    X = jnp.concatenate([erw_ji[:, :32], erw_ki[:, :32], erw_kj[:, :32],
                         hi, hj, hk], axis=1)
    Y = jnp.dot(X, Wbig_ref[...], preferred_element_type=jnp.float32)
    Y = Y + b1_ref[...]
    g = g_ref[...]
    be = be_ref[...]
    subs = []
    for t in range(4):
        y = Y[:, 128 * t:128 * (t + 1)]
        mu = jnp.mean(y, axis=1, keepdims=True)
        var = jnp.mean((y - mu) * (y - mu), axis=1, keepdims=True)
        y = (y - mu) * jax.lax.rsqrt(var + 1e-5) * g[:, 128 * t:128 * (t + 1)]
        y = y + be[:, 128 * t:128 * (t + 1)]
        subs.append(jnp.maximum(y, 0.0))

    Wk2 = Wk2_ref[...]
    Wv2 = Wv2_ref[...]
    k1 = jnp.dot(subs[0], Wk2, preferred_element_type=jnp.float32) + bk2_ref[...]
    k2 = jnp.dot(subs[1], Wk2, preferred_element_type=jnp.float32) + bk2_ref[...]
    v1 = jnp.dot(subs[2], Wv2, preferred_element_type=jnp.float32) + bv2_ref[...]
    v2 = jnp.dot(subs[3], Wv2, preferred_element_type=jnp.float32) + bv2_ref[...]

    q = hqi[:, 128:]
    G = G_ref[...]
    inv = np.float32(1.0 / np.sqrt(8.0))
    e1 = jnp.dot(k1 * q, G, preferred_element_type=jnp.float32) * inv
    e2 = jnp.dot(k2 * q, G, preferred_element_type=jnp.float32) * inv

    ewW = ewW_ref[...]
    w_ji = jax.nn.sigmoid(
        jnp.sum(erw_ji[:, 16:32] * ewW, axis=1, keepdims=True) + ewb_ref[...])
    w_ki = jax.nn.sigmoid(
        jnp.sum(erw_ki[:, 16:32] * ewW, axis=1, keepdims=True) + ewb_ref[...])

    P = (v1[:, :16] + v2[:, :16]) * w_ji * 0.5
    Qv = (v1[:, 16:] + v2[:, 16:]) * w_ki * 0.5
    vcs = []
    for c in range(3):
        rj = erw_ji[:, 32 + c:33 + c]
        rk = erw_ki[:, 32 + c:33 + c]
        vcs.append(P * rj + Qv * rk)

    M1 = jnp.max(e1, axis=0, keepdims=True)
    M2 = jnp.max(e2, axis=0, keepdims=True)
    ex1 = jnp.exp(e1 - M1)
    ex2 = jnp.exp(e2 - M2)

    cols = jnp.concatenate(
        [ex1, ex2,
         ex1 * vcs[0], ex1 * vcs[1], ex1 * vcs[2],
         ex2 * vcs[0], ex2 * vcs[1], ex2 * vcs[2]], axis=1)

    ids_col = idc_ref[...]
    ids_row = idr_ref[...][0]
    Eq = (ids_col == ids_row).astype(jnp.float32)
    Sseg[...] = jnp.dot(Eq, cols, preferred_element_type=jnp.float32)
    Cnt[...] = jnp.sum(Eq, axis=1, keepdims=True)

    Mblk = jnp.concatenate([M1, M2], axis=1)

    def _cond(p):
        return p < B

    def _merge(p):
        n = idc_ref[p, 0]
        srow = Sseg[pl.ds(p, 1), :]
        cnt = Cnt[p, 0].astype(jnp.int32)
        mold = accM[pl.ds(n, 1), :]
        mnew = jnp.maximum(mold, Mblk)
        fa = jnp.exp(mold - mnew)
        fb = jnp.exp(Mblk - mnew)
        sold = accS[pl.ds(n, 1), :]
        accS[pl.ds(n, 1), :] = sold * fa + srow[:, 0:32] * fb
        fa1 = fa[:, 0:16]
        fa2 = fa[:, 16:32]
        fb1 = fb[:, 0:16]
        fb2 = fb[:, 16:32]
        fa96 = jnp.concatenate([fa1, fa1, fa1, fa2, fa2, fa2], axis=1)
        fb96 = jnp.concatenate([fb1, fb1, fb1, fb2, fb2, fb2], axis=1)
        uold = accU[pl.ds(n, 1), :]
        accU[pl.ds(n, 1), :] = uold * fa96 + srow[:, 32:128] * fb96
        accM[pl.ds(n, 1), :] = mnew
        return p + cnt

    jax.lax.while_loop(_cond, _merge, jnp.int32(0))


def _epi_body(accS_ref, accU_ref, out_ref):
    d1 = accS_ref[:, 0:16] + 1e-16
    d2 = accS_ref[:, 16:32] + 1e-16
    outs = []
    for c in range(3):
        t = accU_ref[:, 16 * c:16 * c + 16] / d1 \
            + accU_ref[:, 48 + 16 * c:48 + 16 * c + 16] / d2
        outs.append(jnp.sum(t, axis=1, keepdims=True) * np.float32(0.5 / 16.0))
    out_ref[...] = jnp.concatenate(outs, axis=1)


def _sc_gather(pairs, tc):
    info = pltpu.get_tpu_info().sparse_core
    mesh = plsc.VectorSubcoreMesh(core_axis_name="core",
                                  subcore_axis_name="subcore",
                                  num_cores=info.num_cores,
                                  num_subcores=info.num_subcores)
    tables = []
    tab_ids = []
    for tab, _ in pairs:
        for ti, t in enumerate(tables):
            if t is tab:
                tab_ids.append(ti)
                break
        else:
            tab_ids.append(len(tables))
            tables.append(tab)
    n = len(pairs)
    n_tab = len(tables)
    outs = tuple(jax.ShapeDtypeStruct((tc, tab.shape[1]), tab.dtype)
                 for tab, _ in pairs)

    @functools.partial(pl.kernel, out_type=outs, mesh=mesh)
    def kern(*refs):
        tab_refs = refs[:n_tab]
        idx_refs = refs[n_tab:n_tab + n]
        out_refs = refs[n_tab + n:]

        def body(*bufs):
            ibufs = bufs[:n]
            obufs = bufs[n:]
            for ti, ib, ob in zip(tab_ids, ibufs, obufs):
                pltpu.sync_copy(tab_refs[ti].at[ib.at[0]], ob)

        pltpu.emit_pipeline(
            body,
            grid=(tc // GW,),
            in_specs=[pl.BlockSpec((1, GW), lambda i: (0, i))] * n,
            out_specs=[pl.BlockSpec((GW, tab.shape[1]), lambda i: (i, 0))
                       for tab, _ in pairs],
            core_axis_name="subcore",
            dimension_semantics=(pltpu.PARALLEL,),
        )(*idx_refs, *out_refs)

    res = kern(*tables, *[ix.reshape(1, tc) for _, ix in pairs])
    return tuple(res) if isinstance(res, (tuple, list)) else (res,)


def _perm_ikj(W):
    return jnp.concatenate([W[32:64], W[0:32], W[64:96],
                            W[96:224], W[352:480], W[224:352]], axis=0)


def kernel(h, rel_x, r_feat, edge_feat, edge_index, e_w, id3_i, id3_j, id3_k,
           edgeid_ki, edgeid_ji, edgeid_kj,
           xk_W1, xk_b1, xk_g, xk_be, xk_W2, xk_b2,
           xv_W1, xv_b1, xv_g, xv_be, xv_W2, xv_b2,
           xq_W1, xq_b1, xq_g, xq_be, xq_W2, xq_b2,
           ew_W, ew_b):
    f32 = jnp.float32

    erw = jnp.concatenate(
        [edge_feat, r_feat, rel_x, jnp.zeros((E, EW - 35), f32)], axis=1)

    qtab = pl.pallas_call(
        _q_body,
        grid=(5,),
        in_specs=[pl.BlockSpec((2000, 128), lambda b: (b, 0)),
                  pl.BlockSpec((128, 128), lambda b: (0, 0)),
                  pl.BlockSpec((1, 128), lambda b: (0, 0)),
                  pl.BlockSpec((1, 128), lambda b: (0, 0)),
                  pl.BlockSpec((1, 128), lambda b: (0, 0)),
                  pl.BlockSpec((128, 128), lambda b: (0, 0)),
                  pl.BlockSpec((1, 128), lambda b: (0, 0))],
        out_specs=pl.BlockSpec((2000, 128), lambda b: (b, 0)),
        out_shape=jax.ShapeDtypeStruct((N, 128), f32),
    )(h, xq_W1, xq_b1.reshape(1, 128), xq_g.reshape(1, 128),
      xq_be.reshape(1, 128), xq_W2, xq_b2.reshape(1, 128))

    hq = jnp.concatenate([h, qtab], axis=1)
    c0 = (T // (2 * B)) * B
    chunks = []
    for lo, hi_ in ((0, c0), (c0, T)):
        tc = hi_ - lo
        sl = slice(lo, hi_)
        g_eji, g_eki, g_ekj = _sc_gather(
            [(erw, edgeid_ji[sl]), (erw, edgeid_ki[sl]),
             (erw, edgeid_kj[sl])], tc)
        g_hj, g_hk = _sc_gather([(h, id3_j[sl]), (h, id3_k[sl])], tc)
        (g_hqi,) = _sc_gather([(hq, id3_i[sl])], tc)
        chunks.append((tc, sl, g_eji, g_eki, g_ekj, g_hj, g_hk, g_hqi))

    Wbig = jnp.concatenate(
        [xk_W1, _perm_ikj(xk_W1), xv_W1, _perm_ikj(xv_W1)], axis=1)
    b1 = jnp.concatenate([xk_b1, xk_b1, xv_b1, xv_b1]).reshape(1, 512)
    g512 = jnp.concatenate([xk_g, xk_g, xv_g, xv_g]).reshape(1, 512)
    be512 = jnp.concatenate([xk_be, xk_be, xv_be, xv_be]).reshape(1, 512)
    Ghead = (jnp.arange(128)[:, None] // 8
             == jnp.arange(16)[None, :]).astype(f32)

    bspec_te = pl.BlockSpec((B, EW), lambda b: (b, 0))
    bspec_t128 = pl.BlockSpec((B, 128), lambda b: (b, 0))
    const = lambda shape: pl.BlockSpec(shape, lambda b: tuple(0 for _ in shape))

    acc = None
    for ci, (tc, sl, g_eji, g_eki, g_ekj, g_hj, g_hk, g_hqi) \
            in enumerate(chunks):
        nbc = tc // B
        first = ci == 0
        idc = id3_i[sl].reshape(tc, 1)
        idr = id3_i[sl].reshape(nbc, 1, B)
        in_specs = ([] if first else [const((N, 32)), const((N, 32)),
                                      const((N, 96))]) + [
            pl.BlockSpec((B, 1), lambda b: (b, 0)),
            pl.BlockSpec((1, 1, B), lambda b: (b, 0, 0)),
            bspec_te, bspec_te, bspec_te,
            bspec_t128, bspec_t128,
            pl.BlockSpec((B, 256), lambda b: (b, 0)),
            const((480, 512)), const((1, 512)), const((1, 512)),
            const((1, 512)), const((128, 128)), const((1, 128)),
            const((128, 32)), const((1, 32)), const((1, 16)),
            const((1, 1)), const((128, 16))]
        args = ([] if first else list(acc)) + [
            idc, idr, g_eji, g_eki, g_ekj, g_hj, g_hk, g_hqi,
            Wbig, b1, g512, be512, xk_W2, xk_b2.reshape(1, 128),
            xv_W2, xv_b2.reshape(1, 32), ew_W.reshape(1, 16),
            ew_b.reshape(1, 1), Ghead]
        acc = pl.pallas_call(
            functools.partial(_main_body, first),
            grid=(nbc,),
            in_specs=in_specs,
            out_specs=[pl.BlockSpec((N, 32), lambda b: (0, 0)),
                       pl.BlockSpec((N, 32), lambda b: (0, 0)),
                       pl.BlockSpec((N, 96), lambda b: (0, 0))],
            out_shape=[jax.ShapeDtypeStruct((N, 32), f32),
                       jax.ShapeDtypeStruct((N, 32), f32),
                       jax.ShapeDtypeStruct((N, 96), f32)],
            scratch_shapes=[pltpu.VMEM((B, 128), f32),
                            pltpu.VMEM((B, 1), f32)],
        )(*args)

    out = pl.pallas_call(
        _epi_body,
        out_shape=jax.ShapeDtypeStruct((N, 3), f32),
    )(acc[1], acc[2])

    return out


# speedup vs baseline: 17.6845x; 1.1117x over previous
import functools

import jax
import jax.numpy as jnp
import numpy as np
from jax.experimental import pallas as pl
from jax.experimental.pallas import tpu as pltpu
from jax.experimental.pallas import tpu_sc as plsc

N = 10000
E = 160000
T = 320000
NH = 16
HID = 128
B = 512
NB = T // B
GW = 128
EW = 128


def _q_body(h_ref, W1_ref, b1_ref, g_ref, be_ref, W2_ref, b2_ref, o_ref):
    y = jnp.dot(h_ref[...], W1_ref[...], preferred_element_type=jnp.float32)
    y = y + b1_ref[...]
    mu = jnp.mean(y, axis=1, keepdims=True)
    var = jnp.mean((y - mu) * (y - mu), axis=1, keepdims=True)
    y = (y - mu) * jax.lax.rsqrt(var + 1e-5) * g_ref[...] + be_ref[...]
    y = jnp.maximum(y, 0.0)
    o_ref[...] = jnp.dot(y, W2_ref[...], preferred_element_type=jnp.float32) + b2_ref[...]


def _main_body(first, *refs):
    if first:
        (idc_ref, idr_ref, eji_ref, eki_ref, ekj_ref, hj_ref, hk_ref,
         hqi_ref, Wbig_ref, b1_ref, g_ref, be_ref, Wk2_ref, bk2_ref,
         Wv2_ref, bv2_ref, ewW_ref, ewb_ref, G_ref,
         accM, accS, accU, Sseg, Cnt) = refs
        accMi = accSi = accUi = None
    else:
        (accMi, accSi, accUi, idc_ref, idr_ref, eji_ref, eki_ref, ekj_ref,
         hj_ref, hk_ref, hqi_ref, Wbig_ref, b1_ref, g_ref, be_ref, Wk2_ref,
         bk2_ref, Wv2_ref, bv2_ref, ewW_ref, ewb_ref, G_ref,
         accM, accS, accU, Sseg, Cnt) = refs
    pid = pl.program_id(0)

    @pl.when(pid == 0)
    def _init():
        if first:
            accM[...] = jnp.full((N, 32), -1e30, jnp.float32)
            accS[...] = jnp.zeros((N, 32), jnp.float32)
            accU[...] = jnp.zeros((N, 96), jnp.float32)
        else:
            accM[...] = accMi[...]
            accS[...] = accSi[...]
            accU[...] = accUi[...]

    erw_ji = eji_ref[...]
    erw_ki = eki_ref[...]
    erw_kj = ekj_ref[...]
    hqi = hqi_ref[...]
    hi = hqi[:, :128]
    hj = hj_ref[...]
    hk = hk_ref[...]

    X = jnp.concatenate([erw_ji[:, :32], erw_ki[:, :32], erw_kj[:, :32],
                         hi, hj, hk], axis=1)
    Y = jnp.dot(X, Wbig_ref[...], preferred_element_type=jnp.float32)
    Y = Y + b1_ref[...]
    g = g_ref[...]
    be = be_ref[...]
    subs = []
    for t in range(4):
        y = Y[:, 128 * t:128 * (t + 1)]
        mu = jnp.mean(y, axis=1, keepdims=True)
        var = jnp.mean((y - mu) * (y - mu), axis=1, keepdims=True)
        y = (y - mu) * jax.lax.rsqrt(var + 1e-5) * g[:, 128 * t:128 * (t + 1)]
        y = y + be[:, 128 * t:128 * (t + 1)]
        subs.append(jnp.maximum(y, 0.0))

    Wk2 = Wk2_ref[...]
    Wv2 = Wv2_ref[...]
    k1 = jnp.dot(subs[0], Wk2, preferred_element_type=jnp.float32) + bk2_ref[...]
    k2 = jnp.dot(subs[1], Wk2, preferred_element_type=jnp.float32) + bk2_ref[...]
    v1 = jnp.dot(subs[2], Wv2, preferred_element_type=jnp.float32) + bv2_ref[...]
    v2 = jnp.dot(subs[3], Wv2, preferred_element_type=jnp.float32) + bv2_ref[...]

    q = hqi[:, 128:]
    G = G_ref[...]
    inv = np.float32(1.0 / np.sqrt(8.0))
    e1 = jnp.dot(k1 * q, G, preferred_element_type=jnp.float32) * inv
    e2 = jnp.dot(k2 * q, G, preferred_element_type=jnp.float32) * inv

    ewW = ewW_ref[...]
    w_ji = jax.nn.sigmoid(
        jnp.sum(erw_ji[:, 16:32] * ewW, axis=1, keepdims=True) + ewb_ref[...])
    w_ki = jax.nn.sigmoid(
        jnp.sum(erw_ki[:, 16:32] * ewW, axis=1, keepdims=True) + ewb_ref[...])

    P = (v1[:, :16] + v2[:, :16]) * w_ji * 0.5
    Qv = (v1[:, 16:] + v2[:, 16:]) * w_ki * 0.5
    vcs = []
    for c in range(3):
        rj = erw_ji[:, 32 + c:33 + c]
        rk = erw_ki[:, 32 + c:33 + c]
        vcs.append(P * rj + Qv * rk)

    M1 = jnp.max(e1, axis=0, keepdims=True)
    M2 = jnp.max(e2, axis=0, keepdims=True)
    ex1 = jnp.exp(e1 - M1)
    ex2 = jnp.exp(e2 - M2)

    cols = jnp.concatenate(
        [ex1, ex2,
         ex1 * vcs[0], ex1 * vcs[1], ex1 * vcs[2],
         ex2 * vcs[0], ex2 * vcs[1], ex2 * vcs[2]], axis=1)

    ids_col = idc_ref[...]
    ids_row = idr_ref[...][0]
    Eq = (ids_col == ids_row).astype(jnp.float32)
    Sseg[...] = jnp.dot(Eq, cols, preferred_element_type=jnp.float32)
    Cnt[...] = jnp.sum(Eq, axis=1, keepdims=True)

    Mblk = jnp.concatenate([M1, M2], axis=1)

    def _cond(p):
        return p < B

    def _merge(p):
        n = idc_ref[p, 0]
        srow = Sseg[pl.ds(p, 1), :]
        cnt = Cnt[p, 0].astype(jnp.int32)
        mold = accM[pl.ds(n, 1), :]
        mnew = jnp.maximum(mold, Mblk)
        fa = jnp.exp(mold - mnew)
        fb = jnp.exp(Mblk - mnew)
        sold = accS[pl.ds(n, 1), :]
        accS[pl.ds(n, 1), :] = sold * fa + srow[:, 0:32] * fb
        fa1 = fa[:, 0:16]
        fa2 = fa[:, 16:32]
        fb1 = fb[:, 0:16]
        fb2 = fb[:, 16:32]
        fa96 = jnp.concatenate([fa1, fa1, fa1, fa2, fa2, fa2], axis=1)
        fb96 = jnp.concatenate([fb1, fb1, fb1, fb2, fb2, fb2], axis=1)
        uold = accU[pl.ds(n, 1), :]
        accU[pl.ds(n, 1), :] = uold * fa96 + srow[:, 32:128] * fb96
        accM[pl.ds(n, 1), :] = mnew
        return p + cnt

    jax.lax.while_loop(_cond, _merge, jnp.int32(0))


def _epi_body(accS_ref, accU_ref, out_ref):
    d1 = accS_ref[:, 0:16] + 1e-16
    d2 = accS_ref[:, 16:32] + 1e-16
    outs = []
    for c in range(3):
        t = accU_ref[:, 16 * c:16 * c + 16] / d1 \
            + accU_ref[:, 48 + 16 * c:48 + 16 * c + 16] / d2
        outs.append(jnp.sum(t, axis=1, keepdims=True) * np.float32(0.5 / 16.0))
    out_ref[...] = jnp.concatenate(outs, axis=1)


def _sc_gather(pairs, tc):
    info = pltpu.get_tpu_info().sparse_core
    mesh = plsc.VectorSubcoreMesh(core_axis_name="core",
                                  subcore_axis_name="subcore",
                                  num_cores=info.num_cores,
                                  num_subcores=info.num_subcores)
    tables = []
    tab_ids = []
    for tab, _ in pairs:
        for ti, t in enumerate(tables):
            if t is tab:
                tab_ids.append(ti)
                break
        else:
            tab_ids.append(len(tables))
            tables.append(tab)
    n = len(pairs)
    n_tab = len(tables)
    outs = tuple(jax.ShapeDtypeStruct((tc, tab.shape[1]), tab.dtype)
                 for tab, _ in pairs)

    @functools.partial(pl.kernel, out_type=outs, mesh=mesh)
    def kern(*refs):
        tab_refs = refs[:n_tab]
        idx_refs = refs[n_tab:n_tab + n]
        out_refs = refs[n_tab + n:]

        def body(*bufs):
            ibufs = bufs[:n]
            obufs = bufs[n:]
            for ti, ib, ob in zip(tab_ids, ibufs, obufs):
                pltpu.sync_copy(tab_refs[ti].at[ib.at[0]], ob)

        pltpu.emit_pipeline(
            body,
            grid=(tc // GW,),
            in_specs=[pl.BlockSpec((1, GW), lambda i: (0, i))] * n,
            out_specs=[pl.BlockSpec((GW, tab.shape[1]), lambda i: (i, 0))
                       for tab, _ in pairs],
            core_axis_name="subcore",
            dimension_semantics=(pltpu.PARALLEL,),
        )(*idx_refs, *out_refs)

    res = kern(*tables, *[ix.reshape(1, tc) for _, ix in pairs])
    return tuple(res) if isinstance(res, (tuple, list)) else (res,)


def _perm_ikj(W):
    return jnp.concatenate([W[32:64], W[0:32], W[64:96],
                            W[96:224], W[352:480], W[224:352]], axis=0)


def kernel(h, rel_x, r_feat, edge_feat, edge_index, e_w, id3_i, id3_j, id3_k,
           edgeid_ki, edgeid_ji, edgeid_kj,
           xk_W1, xk_b1, xk_g, xk_be, xk_W2, xk_b2,
           xv_W1, xv_b1, xv_g, xv_be, xv_W2, xv_b2,
           xq_W1, xq_b1, xq_g, xq_be, xq_W2, xq_b2,
           ew_W, ew_b):
    f32 = jnp.float32

    erw = jnp.concatenate(
        [edge_feat, r_feat, rel_x, jnp.zeros((E, EW - 35), f32)], axis=1)

    qtab = pl.pallas_call(
        _q_body,
        grid=(5,),
        in_specs=[pl.BlockSpec((2000, 128), lambda b: (b, 0)),
                  pl.BlockSpec((128, 128), lambda b: (0, 0)),
                  pl.BlockSpec((1, 128), lambda b: (0, 0)),
                  pl.BlockSpec((1, 128), lambda b: (0, 0)),
                  pl.BlockSpec((1, 128), lambda b: (0, 0)),
                  pl.BlockSpec((128, 128), lambda b: (0, 0)),
                  pl.BlockSpec((1, 128), lambda b: (0, 0))],
        out_specs=pl.BlockSpec((2000, 128), lambda b: (b, 0)),
        out_shape=jax.ShapeDtypeStruct((N, 128), f32),
    )(h, xq_W1, xq_b1.reshape(1, 128), xq_g.reshape(1, 128),
      xq_be.reshape(1, 128), xq_W2, xq_b2.reshape(1, 128))

    hq = jnp.concatenate([h, qtab], axis=1)
    n_chunks = 4
    bounds = [(i * T // n_chunks) // B * B for i in range(n_chunks)] + [T]
    chunks = []
    for lo, hi_ in zip(bounds[:-1], bounds[1:]):
        tc = hi_ - lo
        sl = slice(lo, hi_)
        g_eji, g_eki, g_ekj = _sc_gather(
            [(erw, edgeid_ji[sl]), (erw, edgeid_ki[sl]),
             (erw, edgeid_kj[sl])], tc)
        g_hj, g_hk = _sc_gather([(h, id3_j[sl]), (h, id3_k[sl])], tc)
        (g_hqi,) = _sc_gather([(hq, id3_i[sl])], tc)
        chunks.append((tc, sl, g_eji, g_eki, g_ekj, g_hj, g_hk, g_hqi))

    Wbig = jnp.concatenate(
        [xk_W1, _perm_ikj(xk_W1), xv_W1, _perm_ikj(xv_W1)], axis=1)
    b1 = jnp.concatenate([xk_b1, xk_b1, xv_b1, xv_b1]).reshape(1, 512)
    g512 = jnp.concatenate([xk_g, xk_g, xv_g, xv_g]).reshape(1, 512)
    be512 = jnp.concatenate([xk_be, xk_be, xv_be, xv_be]).reshape(1, 512)
    Ghead = (jnp.arange(128)[:, None] // 8
             == jnp.arange(16)[None, :]).astype(f32)

    bspec_te = pl.BlockSpec((B, EW), lambda b: (b, 0))
    bspec_t128 = pl.BlockSpec((B, 128), lambda b: (b, 0))
    const = lambda shape: pl.BlockSpec(shape, lambda b: tuple(0 for _ in shape))

    acc = None
    for ci, (tc, sl, g_eji, g_eki, g_ekj, g_hj, g_hk, g_hqi) \
            in enumerate(chunks):
        nbc = tc // B
        first = ci == 0
        idc = id3_i[sl].reshape(tc, 1)
        idr = id3_i[sl].reshape(nbc, 1, B)
        in_specs = ([] if first else [const((N, 32)), const((N, 32)),
                                      const((N, 96))]) + [
            pl.BlockSpec((B, 1), lambda b: (b, 0)),
            pl.BlockSpec((1, 1, B), lambda b: (b, 0, 0)),
            bspec_te, bspec_te, bspec_te,
            bspec_t128, bspec_t128,
            pl.BlockSpec((B, 256), lambda b: (b, 0)),
            const((480, 512)), const((1, 512)), const((1, 512)),
            const((1, 512)), const((128, 128)), const((1, 128)),
            const((128, 32)), const((1, 32)), const((1, 16)),
            const((1, 1)), const((128, 16))]
        args = ([] if first else list(acc)) + [
            idc, idr, g_eji, g_eki, g_ekj, g_hj, g_hk, g_hqi,
            Wbig, b1, g512, be512, xk_W2, xk_b2.reshape(1, 128),
            xv_W2, xv_b2.reshape(1, 32), ew_W.reshape(1, 16),
            ew_b.reshape(1, 1), Ghead]
        acc = pl.pallas_call(
            functools.partial(_main_body, first),
            grid=(nbc,),
            in_specs=in_specs,
            out_specs=[pl.BlockSpec((N, 32), lambda b: (0, 0)),
                       pl.BlockSpec((N, 32), lambda b: (0, 0)),
                       pl.BlockSpec((N, 96), lambda b: (0, 0))],
            out_shape=[jax.ShapeDtypeStruct((N, 32), f32),
                       jax.ShapeDtypeStruct((N, 32), f32),
                       jax.ShapeDtypeStruct((N, 96), f32)],
            scratch_shapes=[pltpu.VMEM((B, 128), f32),
                            pltpu.VMEM((B, 1), f32)],
        )(*args)

    out = pl.pallas_call(
        _epi_body,
        out_shape=jax.ShapeDtypeStruct((N, 3), f32),
    )(acc[1], acc[2])

    return out
